# baseline (speedup 1.0000x reference)
import jax
import jax.numpy as jnp
from jax.experimental import pallas as pl

B = 16384
H = 64
OUT = 64
NB = 4
LN_EPS = 1e-5
BN_EPS = 1e-5
TILE = 1024


def _body(x_ref, ln_w_ref, ln_b_ref, w1_ref, b1_ref, w2_ref, b2_ref,
          fc_w_ref, fc_b_ref, o_ref):
    x = x_ref[...]
    for i in range(NB):
        res = x
        mu = jnp.mean(x, axis=-1, keepdims=True)
        xc = x - mu
        var = jnp.mean(xc * xc, axis=-1, keepdims=True)
        h = xc * jax.lax.rsqrt(var + LN_EPS)
        h = h * ln_w_ref[i, :][None, :] + ln_b_ref[i, :][None, :]
        h = jnp.dot(h, w1_ref[i], preferred_element_type=jnp.float32)
        h = h + b1_ref[i, :][None, :]
        h = jnp.maximum(h, 0.0)
        h = jnp.dot(h, w2_ref[i], preferred_element_type=jnp.float32)
        h = h + b2_ref[i, :][None, :]
        x = h + res
    o = jnp.dot(x, fc_w_ref[...], preferred_element_type=jnp.float32)
    o_ref[...] = o + fc_b_ref[0, :][None, :]


def kernel(x, ln_w, ln_b, w1, b1, bn_w, bn_b, w2, b2, fc_w, fc_b):
    g = bn_w * (1.0 / jnp.sqrt(1.0 + BN_EPS))
    w1f = w1 * g[:, None, :]
    b1f = b1 * g + bn_b
    fc_b2 = fc_b.reshape(1, OUT)

    return pl.pallas_call(
        _body,
        grid=(B // TILE,),
        in_specs=[
            pl.BlockSpec((TILE, H), lambda i: (i, 0)),
            pl.BlockSpec((NB, H), lambda i: (0, 0)),
            pl.BlockSpec((NB, H), lambda i: (0, 0)),
            pl.BlockSpec((NB, H, H), lambda i: (0, 0, 0)),
            pl.BlockSpec((NB, H), lambda i: (0, 0)),
            pl.BlockSpec((NB, H, H), lambda i: (0, 0, 0)),
            pl.BlockSpec((NB, H), lambda i: (0, 0)),
            pl.BlockSpec((H, OUT), lambda i: (0, 0)),
            pl.BlockSpec((1, OUT), lambda i: (0, 0)),
        ],
        out_specs=pl.BlockSpec((TILE, OUT), lambda i: (i, 0)),
        out_shape=jax.ShapeDtypeStruct((B, OUT), jnp.float32),
    )(x, ln_w, ln_b, w1f, b1f, w2, b2, fc_w, fc_b2)


# speedup vs baseline: 1.0320x; 1.0320x over previous
import jax
import jax.numpy as jnp
from jax.experimental import pallas as pl

B = 16384
H = 64
OUT = 64
NB = 4
LN_EPS = 1e-5
BN_EPS = 1e-5
P = 2 * H
BP = B // 2
TILE = 1024


def _body(x_ref, m_ref, w1_ref, c_ref, d_ref, w2_ref, b2_ref,
          fc_w_ref, fc_b_ref, o_ref):
    x = x_ref[...]
    m = m_ref[...]
    for i in range(NB):
        mu = jnp.dot(x, m, preferred_element_type=jnp.float32)
        q = jnp.dot(x * x, m, preferred_element_type=jnp.float32)
        t = jnp.dot(x, w1_ref[i], preferred_element_type=jnp.float32)
        r = jax.lax.rsqrt(q - mu * mu + LN_EPS)
        h = r * (t - mu * c_ref[i, :][None, :]) + d_ref[i, :][None, :]
        h = jnp.maximum(h, 0.0)
        h = jnp.dot(h, w2_ref[i], preferred_element_type=jnp.float32)
        x = x + h + b2_ref[i, :][None, :]
    o = jnp.dot(x, fc_w_ref[...], preferred_element_type=jnp.float32)
    o_ref[...] = o + fc_b_ref[0, :][None, :]


def _bdiag(w):
    z = jnp.zeros(w.shape, w.dtype)
    top = jnp.concatenate([w, z], axis=-1)
    bot = jnp.concatenate([z, w], axis=-1)
    return jnp.concatenate([top, bot], axis=-2)


def kernel(x, ln_w, ln_b, w1, b1, bn_w, bn_b, w2, b2, fc_w, fc_b):
    g = bn_w * (1.0 / jnp.sqrt(1.0 + BN_EPS))
    w1g = (ln_w[:, :, None] * w1) * g[:, None, :]
    d = jnp.einsum("nh,nho->no", ln_b, w1 * g[:, None, :]) + b1 * g + bn_b

    w1p = _bdiag(w1g)
    w2p = _bdiag(w2)
    fcp = _bdiag(fc_w)
    mp = _bdiag(jnp.full((H, H), 1.0 / H, jnp.float32))

    c = jnp.sum(w1p, axis=1)
    d2 = jnp.tile(d, (1, 2))
    b2p = jnp.tile(b2, (1, 2))
    fcbp = jnp.tile(fc_b.reshape(1, OUT), (1, 2))

    xp = x.reshape(BP, P)

    out = pl.pallas_call(
        _body,
        grid=(BP // TILE,),
        in_specs=[
            pl.BlockSpec((TILE, P), lambda i: (i, 0)),
            pl.BlockSpec((P, P), lambda i: (0, 0)),
            pl.BlockSpec((NB, P, P), lambda i: (0, 0, 0)),
            pl.BlockSpec((NB, P), lambda i: (0, 0)),
            pl.BlockSpec((NB, P), lambda i: (0, 0)),
            pl.BlockSpec((NB, P, P), lambda i: (0, 0, 0)),
            pl.BlockSpec((NB, P), lambda i: (0, 0)),
            pl.BlockSpec((P, P), lambda i: (0, 0)),
            pl.BlockSpec((1, P), lambda i: (0, 0)),
        ],
        out_specs=pl.BlockSpec((TILE, P), lambda i: (i, 0)),
        out_shape=jax.ShapeDtypeStruct((BP, P), jnp.float32),
    )(xp, mp, w1p, c, d2, w2p, b2p, fcp, fcbp)
    return out.reshape(B, OUT)


# speedup vs baseline: 1.1171x; 1.0824x over previous
import jax
import jax.numpy as jnp
from jax.experimental import pallas as pl
from jax.experimental.pallas import tpu as pltpu

B = 16384
H = 64
OUT = 64
NB = 4
LN_EPS = 1e-5
BN_EPS = 1e-5
P = 2 * H
BP = B // 2
TILE = 1024


def _body(x_ref, lnw_ref, lnb_ref, w1_ref, b1_ref, bnw_ref, bnb_ref,
          w2_ref, b2_ref, fcw_ref, fcb_ref, o_ref, wp_ref, vec_ref):
    lane = jax.lax.broadcasted_iota(jnp.int32, (P, P), 1)
    sub = jax.lax.broadcasted_iota(jnp.int32, (P, P), 0)

    @pl.when(pl.program_id(0) == 0)
    def _setup():
        bmask = (((lane ^ sub) & H) == 0).astype(jnp.float32)
        dmask = (lane == sub).astype(jnp.float32)

        def bdiag(w):
            t = jnp.concatenate([w, w], axis=1)
            return jnp.concatenate([t, t], axis=0) * bmask

        def diag(v):
            return jnp.broadcast_to(v, (P, P)) * dmask

        def tile2(v):
            return jnp.concatenate([v, v], axis=1)

        wp_ref[NB + NB + 1] = bmask * (1.0 / H)
        wp_ref[NB + NB] = bdiag(fcw_ref[...])
        vec_ref[3 * NB:3 * NB + 1, :] = tile2(fcb_ref[...])

        s = 1.0 / jnp.sqrt(1.0 + BN_EPS)
        for i in range(NB):
            gp = tile2(bnw_ref[i:i + 1, :]) * s
            w1p0 = bdiag(w1_ref[i])
            w1g = jnp.dot(diag(tile2(lnw_ref[i:i + 1, :])),
                          jnp.dot(w1p0, diag(gp),
                                  preferred_element_type=jnp.float32),
                          preferred_element_type=jnp.float32)
            wp_ref[i] = w1g
            wp_ref[NB + i] = bdiag(w2_ref[i])
            vec_ref[i:i + 1, :] = jnp.sum(w1g, axis=0, keepdims=True)
            d = jnp.dot(tile2(lnb_ref[i:i + 1, :]),
                        jnp.dot(w1p0, diag(gp),
                                preferred_element_type=jnp.float32),
                        preferred_element_type=jnp.float32)
            d = d + tile2(b1_ref[i:i + 1, :]) * gp + tile2(bnb_ref[i:i + 1, :])
            vec_ref[NB + i:NB + i + 1, :] = d
            vec_ref[2 * NB + i:2 * NB + i + 1, :] = tile2(b2_ref[i:i + 1, :])

    x = x_ref[...]
    m = wp_ref[NB + NB + 1]
    for i in range(NB):
        mu = jnp.dot(x, m, preferred_element_type=jnp.float32)
        q = jnp.dot(x * x, m, preferred_element_type=jnp.float32)
        t = jnp.dot(x, wp_ref[i], preferred_element_type=jnp.float32)
        r = jax.lax.rsqrt(q - mu * mu + LN_EPS)
        h = r * (t - mu * vec_ref[i:i + 1, :]) + vec_ref[NB + i:NB + i + 1, :]
        h = jnp.maximum(h, 0.0)
        h = jnp.dot(h, wp_ref[NB + i], preferred_element_type=jnp.float32)
        x = x + h + vec_ref[2 * NB + i:2 * NB + i + 1, :]
    o = jnp.dot(x, wp_ref[NB + NB], preferred_element_type=jnp.float32)
    o_ref[...] = o + vec_ref[3 * NB:3 * NB + 1, :]


def kernel(x, ln_w, ln_b, w1, b1, bn_w, bn_b, w2, b2, fc_w, fc_b):
    xp = x.reshape(BP, P)
    full = lambda *shape: pl.BlockSpec(shape, lambda i: (0,) * len(shape))
    out = pl.pallas_call(
        _body,
        grid=(BP // TILE,),
        in_specs=[
            pl.BlockSpec((TILE, P), lambda i: (i, 0)),
            full(NB, H), full(NB, H), full(NB, H, H), full(NB, H),
            full(NB, H), full(NB, H), full(NB, H, H), full(NB, H),
            full(H, OUT), full(1, OUT),
        ],
        out_specs=pl.BlockSpec((TILE, P), lambda i: (i, 0)),
        out_shape=jax.ShapeDtypeStruct((BP, P), jnp.float32),
        scratch_shapes=[
            pltpu.VMEM((2 * NB + 2, P, P), jnp.float32),
            pltpu.VMEM((3 * NB + 1, P), jnp.float32),
        ],
    )(xp, ln_w, ln_b, w1, b1, bn_w, bn_b, w2, b2, fc_w,
      fc_b.reshape(1, OUT))
    return out.reshape(B, OUT)


# speedup vs baseline: 1.7150x; 1.5352x over previous
import jax
import jax.numpy as jnp
from jax.experimental import pallas as pl
from jax.experimental.pallas import tpu as pltpu

B = 16384
H = 64
OUT = 64
NB = 4
LN_EPS = 1e-5
BN_EPS = 1e-5
P = 2 * H
BP = B // 2
TILE = 2048


def _body(x_ref, lnw_ref, lnb_ref, w1_ref, b1_ref, bnw_ref, bnb_ref,
          w2_ref, b2_ref, fcw_ref, fcb_ref, o_ref, wp_ref, vec_ref):
    lane = jax.lax.broadcasted_iota(jnp.int32, (P, P), 1)
    sub = jax.lax.broadcasted_iota(jnp.int32, (P, P), 0)

    @pl.when(pl.program_id(0) == 0)
    def _setup():
        bmask = (((lane ^ sub) & H) == 0).astype(jnp.float32)
        dmask = (lane == sub).astype(jnp.float32)

        def bdiag(w):
            t = jnp.concatenate([w, w], axis=1)
            return jnp.concatenate([t, t], axis=0) * bmask

        def diag(v):
            return jnp.broadcast_to(v, (P, P)) * dmask

        def tile2(v):
            return jnp.concatenate([v, v], axis=1)

        wp_ref[NB + NB + 1] = bmask * (1.0 / H)
        wp_ref[NB + NB] = bdiag(fcw_ref[...])
        vec_ref[3 * NB:3 * NB + 1, :] = tile2(fcb_ref[...])

        s = 1.0 / jnp.sqrt(1.0 + BN_EPS)
        for i in range(NB):
            gp = tile2(bnw_ref[i:i + 1, :]) * s
            w1p0 = bdiag(w1_ref[i])
            w1g = jnp.dot(diag(tile2(lnw_ref[i:i + 1, :])),
                          jnp.dot(w1p0, diag(gp),
                                  preferred_element_type=jnp.float32),
                          preferred_element_type=jnp.float32)
            wp_ref[i] = w1g
            wp_ref[NB + i] = bdiag(w2_ref[i])
            vec_ref[i:i + 1, :] = jnp.sum(w1g, axis=0, keepdims=True)
            d = jnp.dot(tile2(lnb_ref[i:i + 1, :]),
                        jnp.dot(w1p0, diag(gp),
                                preferred_element_type=jnp.float32),
                        preferred_element_type=jnp.float32)
            d = d + tile2(b1_ref[i:i + 1, :]) * gp + tile2(bnb_ref[i:i + 1, :])
            vec_ref[NB + i:NB + i + 1, :] = d
            vec_ref[2 * NB + i:2 * NB + i + 1, :] = tile2(b2_ref[i:i + 1, :])

    x = jnp.concatenate([x_ref[0:TILE, :], x_ref[TILE:2 * TILE, :]], axis=1)
    m = wp_ref[NB + NB + 1]
    for i in range(NB):
        mu = jnp.dot(x, m, preferred_element_type=jnp.float32)
        q = jnp.dot(x * x, m, preferred_element_type=jnp.float32)
        t = jnp.dot(x, wp_ref[i], preferred_element_type=jnp.float32)
        r = jax.lax.rsqrt(q - mu * mu + LN_EPS)
        h = r * (t - mu * vec_ref[i:i + 1, :]) + vec_ref[NB + i:NB + i + 1, :]
        h = jnp.maximum(h, 0.0)
        h = jnp.dot(h, wp_ref[NB + i], preferred_element_type=jnp.float32)
        x = x + h + vec_ref[2 * NB + i:2 * NB + i + 1, :]
    o = jnp.dot(x, wp_ref[NB + NB], preferred_element_type=jnp.float32)
    o = o + vec_ref[3 * NB:3 * NB + 1, :]
    o_ref[0:TILE, :] = o[:, 0:H]
    o_ref[TILE:2 * TILE, :] = o[:, H:P]


def kernel(x, ln_w, ln_b, w1, b1, bn_w, bn_b, w2, b2, fc_w, fc_b):
    full = lambda *shape: pl.BlockSpec(shape, lambda i: (0,) * len(shape))
    return pl.pallas_call(
        _body,
        grid=(BP // TILE,),
        in_specs=[
            pl.BlockSpec((2 * TILE, H), lambda i: (i, 0)),
            full(NB, H), full(NB, H), full(NB, H, H), full(NB, H),
            full(NB, H), full(NB, H), full(NB, H, H), full(NB, H),
            full(H, OUT), full(1, OUT),
        ],
        out_specs=pl.BlockSpec((2 * TILE, H), lambda i: (i, 0)),
        out_shape=jax.ShapeDtypeStruct((B, OUT), jnp.float32),
        scratch_shapes=[
            pltpu.VMEM((2 * NB + 2, P, P), jnp.float32),
            pltpu.VMEM((3 * NB + 1, P), jnp.float32),
        ],
    )(x, ln_w, ln_b, w1, b1, bn_w, bn_b, w2, b2, fc_w,
      fc_b.reshape(1, OUT))


# speedup vs baseline: 1.7985x; 1.0487x over previous
import jax
import jax.numpy as jnp
from jax.experimental import pallas as pl
from jax.experimental.pallas import tpu as pltpu

B = 16384
H = 64
OUT = 64
NB = 4
LN_EPS = 1e-5
BN_EPS = 1e-5
P = 2 * H
BP = B // 2
TILE = 2048


def _body(x_ref, lnw_ref, lnb_ref, w1_ref, b1_ref, bnw_ref, bnb_ref,
          w2_ref, b2_ref, fcw_ref, fcb_ref, o_ref, w1m_ref, wp_ref, vec_ref):
    lane = jax.lax.broadcasted_iota(jnp.int32, (P, P), 1)
    sub = jax.lax.broadcasted_iota(jnp.int32, (P, P), 0)

    @pl.when(pl.program_id(0) == 0)
    def _setup():
        bmask = (((lane ^ sub) & H) == 0).astype(jnp.float32)
        dmask = (lane == sub).astype(jnp.float32)
        m = bmask * (1.0 / H)

        def bdiag(w):
            t = jnp.concatenate([w, w], axis=1)
            return jnp.concatenate([t, t], axis=0) * bmask

        def diag(v):
            return jnp.broadcast_to(v, (P, P)) * dmask

        def tile2(v):
            return jnp.concatenate([v, v], axis=1)

        wp_ref[NB] = bdiag(fcw_ref[...])
        vec_ref[2 * NB:2 * NB + 1, :] = tile2(fcb_ref[...])

        s = 1.0 / jnp.sqrt(1.0 + BN_EPS)
        for i in range(NB):
            gp = tile2(bnw_ref[i:i + 1, :]) * s
            w1p0g = jnp.dot(bdiag(w1_ref[i]), diag(gp),
                            preferred_element_type=jnp.float32)
            w1g = jnp.dot(diag(tile2(lnw_ref[i:i + 1, :])), w1p0g,
                          preferred_element_type=jnp.float32)
            c = jnp.sum(w1g, axis=0, keepdims=True)
            w1m_ref[i] = jnp.concatenate([w1g - m * c, m], axis=1)
            wp_ref[i] = bdiag(w2_ref[i])
            d = jnp.dot(tile2(lnb_ref[i:i + 1, :]), w1p0g,
                        preferred_element_type=jnp.float32)
            d = d + tile2(b1_ref[i:i + 1, :]) * gp + tile2(bnb_ref[i:i + 1, :])
            vec_ref[i:i + 1, :] = d
            vec_ref[NB + i:NB + i + 1, :] = tile2(b2_ref[i:i + 1, :])

    x = jnp.concatenate([x_ref[0:TILE, :], x_ref[TILE:2 * TILE, :]], axis=1)
    m = w1m_ref[0][:, P:2 * P]
    for i in range(NB):
        tm = jnp.dot(x, w1m_ref[i], preferred_element_type=jnp.float32)
        q = jnp.dot(x * x, m, preferred_element_type=jnp.float32)
        t = tm[:, 0:P]
        mu = tm[:, P:2 * P]
        r = jax.lax.rsqrt(q - mu * mu + LN_EPS)
        h = r * t + vec_ref[i:i + 1, :]
        h = jnp.maximum(h, 0.0)
        h = jnp.dot(h, wp_ref[i], preferred_element_type=jnp.float32)
        x = x + h + vec_ref[NB + i:NB + i + 1, :]
    o = jnp.dot(x, wp_ref[NB], preferred_element_type=jnp.float32)
    o = o + vec_ref[2 * NB:2 * NB + 1, :]
    o_ref[0:TILE, :] = o[:, 0:H]
    o_ref[TILE:2 * TILE, :] = o[:, H:P]


def kernel(x, ln_w, ln_b, w1, b1, bn_w, bn_b, w2, b2, fc_w, fc_b):
    full = lambda *shape: pl.BlockSpec(shape, lambda i: (0,) * len(shape))
    return pl.pallas_call(
        _body,
        grid=(BP // TILE,),
        in_specs=[
            pl.BlockSpec((2 * TILE, H), lambda i: (i, 0)),
            full(NB, H), full(NB, H), full(NB, H, H), full(NB, H),
            full(NB, H), full(NB, H), full(NB, H, H), full(NB, H),
            full(H, OUT), full(1, OUT),
        ],
        out_specs=pl.BlockSpec((2 * TILE, H), lambda i: (i, 0)),
        out_shape=jax.ShapeDtypeStruct((B, OUT), jnp.float32),
        scratch_shapes=[
            pltpu.VMEM((NB, P, 2 * P), jnp.float32),
            pltpu.VMEM((NB + 1, P, P), jnp.float32),
            pltpu.VMEM((2 * NB + 1, P), jnp.float32),
        ],
    )(x, ln_w, ln_b, w1, b1, bn_w, bn_b, w2, b2, fc_w,
      fc_b.reshape(1, OUT))


# speedup vs baseline: 1.8050x; 1.0036x over previous
import jax
import jax.numpy as jnp
from jax.experimental import pallas as pl
from jax.experimental.pallas import tpu as pltpu

B = 16384
H = 64
OUT = 64
NB = 4
LN_EPS = 1e-5
BN_EPS = 1e-5
P = 2 * H
BP = B // 2
TILE = 2048


def _body(x_ref, xb_ref, lnw_ref, lnb_ref, w1_ref, b1_ref, bnw_ref, bnb_ref,
          w2_ref, b2_ref, fcw_ref, fcb_ref, o_ref, w1m_ref, wp_ref, vec_ref):
    lane = jax.lax.broadcasted_iota(jnp.int32, (P, P), 1)
    sub = jax.lax.broadcasted_iota(jnp.int32, (P, P), 0)

    @pl.when(pl.program_id(0) == 0)
    def _setup():
        bmask = (((lane ^ sub) & H) == 0).astype(jnp.float32)
        dmask = (lane == sub).astype(jnp.float32)
        m = bmask * (1.0 / H)

        def bdiag(w):
            t = jnp.concatenate([w, w], axis=1)
            return jnp.concatenate([t, t], axis=0) * bmask

        def diag(v):
            return jnp.broadcast_to(v, (P, P)) * dmask

        def tile2(v):
            return jnp.concatenate([v, v], axis=1)

        wp_ref[NB] = bdiag(fcw_ref[...])
        vec_ref[2 * NB:2 * NB + 1, :] = tile2(fcb_ref[...])

        s = 1.0 / jnp.sqrt(1.0 + BN_EPS)
        for i in range(NB):
            gp = tile2(bnw_ref[i:i + 1, :]) * s
            w1p0g = jnp.dot(bdiag(w1_ref[i]), diag(gp),
                            preferred_element_type=jnp.float32)
            w1g = jnp.dot(diag(tile2(lnw_ref[i:i + 1, :])), w1p0g,
                          preferred_element_type=jnp.float32)
            c = jnp.sum(w1g, axis=0, keepdims=True)
            w1m_ref[i] = jnp.concatenate([w1g - m * c, m], axis=1)
            wp_ref[i] = bdiag(w2_ref[i])
            d = jnp.dot(tile2(lnb_ref[i:i + 1, :]), w1p0g,
                        preferred_element_type=jnp.float32)
            d = d + tile2(b1_ref[i:i + 1, :]) * gp + tile2(bnb_ref[i:i + 1, :])
            vec_ref[i:i + 1, :] = d
            vec_ref[NB + i:NB + i + 1, :] = tile2(b2_ref[i:i + 1, :])

    x = jnp.concatenate([x_ref[0], xb_ref[0]], axis=1)
    m = w1m_ref[0][:, P:2 * P]
    for i in range(NB):
        tm = jnp.dot(x, w1m_ref[i], preferred_element_type=jnp.float32)
        q = jnp.dot(x * x, m, preferred_element_type=jnp.float32)
        t = tm[:, 0:P]
        mu = tm[:, P:2 * P]
        r = jax.lax.rsqrt(q - mu * mu + LN_EPS)
        h = r * t + vec_ref[i:i + 1, :]
        h = jnp.maximum(h, 0.0)
        h = jnp.dot(h, wp_ref[i], preferred_element_type=jnp.float32)
        x = x + h + vec_ref[NB + i:NB + i + 1, :]
    o = jnp.dot(x, wp_ref[NB], preferred_element_type=jnp.float32)
    o = o + vec_ref[2 * NB:2 * NB + 1, :]
    o_ref[0] = o[:, 0:H]
    o_ref[1] = o[:, H:P]


def kernel(x, ln_w, ln_b, w1, b1, bn_w, bn_b, w2, b2, fc_w, fc_b):
    full = lambda *shape: pl.BlockSpec(shape, lambda i: (0,) * len(shape))
    x3 = x.reshape(2, BP, H)
    out = pl.pallas_call(
        _body,
        grid=(BP // TILE,),
        in_specs=[
            pl.BlockSpec((1, TILE, H), lambda i: (0, i, 0)),
            pl.BlockSpec((1, TILE, H), lambda i: (1, i, 0)),
            full(NB, H), full(NB, H), full(NB, H, H), full(NB, H),
            full(NB, H), full(NB, H), full(NB, H, H), full(NB, H),
            full(H, OUT), full(1, OUT),
        ],
        out_specs=pl.BlockSpec((2, TILE, H), lambda i: (0, i, 0)),
        out_shape=jax.ShapeDtypeStruct((2, BP, OUT), jnp.float32),
        scratch_shapes=[
            pltpu.VMEM((NB, P, 2 * P), jnp.float32),
            pltpu.VMEM((NB + 1, P, P), jnp.float32),
            pltpu.VMEM((2 * NB + 1, P), jnp.float32),
        ],
    )(x3, x3, ln_w, ln_b, w1, b1, bn_w, bn_b, w2, b2, fc_w,
      fc_b.reshape(1, OUT))
    return out.reshape(B, OUT)
